# baseline (speedup 1.0000x reference)
import functools

import jax
import jax.numpy as jnp
from jax import lax
from jax.experimental import pallas as pl
from jax.experimental.pallas import tpu as pltpu
from jax.experimental.pallas import tpu_sc as plsc

N = 10000
E = 320000
B = 64
DF = 128
DE = 16
DU = 32
H = 128

NC = 2
NS = 16
NW = NC * NS
DFP = DF // 2
SZ = 128
NCHUNK = E // SZ


def _tabs_body(x_ref, batch_ref, u_ref, w_src_ref, w_dst_ref, w_u_ref, be1_ref,
               a_ref, b_ref):
    x = x_ref[...]
    u1 = jnp.dot(u_ref[...], w_u_ref[...], preferred_element_type=jnp.float32)
    oh = (batch_ref[...] == lax.broadcasted_iota(jnp.int32, (N, B), 1)
          ).astype(jnp.float32)
    a_ref[...] = (jnp.dot(x, w_src_ref[...], preferred_element_type=jnp.float32)
                  + jnp.dot(oh, u1, preferred_element_type=jnp.float32)
                  + be1_ref[...])
    b_ref[...] = jnp.dot(x, w_dst_ref[...], preferred_element_type=jnp.float32)


def _tabs(x, batch2d, u, w_src, w_dst, w_u, be1):
    return pl.pallas_call(
        _tabs_body,
        out_shape=[jax.ShapeDtypeStruct((N, DF), jnp.float32),
                   jax.ShapeDtypeStruct((N, DF), jnp.float32)],
    )(x, batch2d, u, w_src, w_dst, w_u, be1)


SZ2 = 40
EPT = E // NW
CPT = EPT // SZ2


def _gather_body(a_hbm, b_hbm, src_hbm, dest_hbm, g_hbm,
                 ia0, ib0, ia1, ib1, ra0, rb0, ra1, rb1, ob0, ob1,
                 sa0, sb0, sa1, sb1, so0, so1):
    cid = lax.axis_index("c")
    sid = lax.axis_index("s")
    wid = sid * NC + cid
    base = wid * EPT

    def load_start(j, ia, ib, ra, rb, sa, sb):
        off = base + j * SZ2
        pltpu.sync_copy(src_hbm.at[pl.ds(off, SZ2)], ia)
        pltpu.sync_copy(dest_hbm.at[pl.ds(off, SZ2)], ib)
        pltpu.async_copy(a_hbm.at[ia], ra, sa)
        pltpu.async_copy(b_hbm.at[ib], rb, sb)

    def wait_gathers(ia, ib, ra, rb, sa, sb):
        pltpu.make_async_copy(a_hbm.at[ia], ra, sa).wait()
        pltpu.make_async_copy(b_hbm.at[ib], rb, sb).wait()

    def add_into(ra, rb, ob):
        @plsc.parallel_loop(0, SZ2, unroll=4)
        def _(i):
            for k in range(DFP // 16):
                sl = pl.ds(k * 16, 16)
                a = plsc.bitcast(ra[i, sl], jnp.bfloat16)
                b = plsc.bitcast(rb[i, sl], jnp.bfloat16)
                ob[i, sl] = plsc.bitcast(a + b, jnp.float32)

    def out_start(j, ob, so):
        pltpu.async_copy(ob, g_hbm.at[pl.ds(base + j * SZ2, SZ2)], so)

    def out_wait(j, ob, so):
        pltpu.make_async_copy(ob, g_hbm.at[pl.ds(base + j * SZ2, SZ2)],
                              so).wait()

    load_start(0, ia0, ib0, ra0, rb0, sa0, sb0)

    @pl.loop(0, CPT // 2)
    def _(jj):
        j0 = 2 * jj
        j1 = j0 + 1
        load_start(j1, ia1, ib1, ra1, rb1, sa1, sb1)
        wait_gathers(ia0, ib0, ra0, rb0, sa0, sb0)

        @pl.when(jj > 0)
        def _():
            out_wait(j0 - 2, ob0, so0)
        add_into(ra0, rb0, ob0)
        out_start(j0, ob0, so0)

        @pl.when(jj < CPT // 2 - 1)
        def _():
            load_start(j0 + 2, ia0, ib0, ra0, rb0, sa0, sb0)
        wait_gathers(ia1, ib1, ra1, rb1, sa1, sb1)

        @pl.when(jj > 0)
        def _():
            out_wait(j1 - 2, ob1, so1)
        add_into(ra1, rb1, ob1)
        out_start(j1, ob1, so1)

    out_wait(CPT - 2, ob0, so0)
    out_wait(CPT - 1, ob1, so1)


def _gather(a_tab, b_tab, src, dest):
    mesh = plsc.VectorSubcoreMesh(core_axis_name="c", subcore_axis_name="s")
    f32 = jnp.float32
    return pl.kernel(
        _gather_body,
        out_type=jax.ShapeDtypeStruct((E, DFP), f32),
        mesh=mesh,
        compiler_params=pltpu.CompilerParams(needs_layout_passes=False,
                                             use_tc_tiling_on_sc=False),
        scratch_types=[
            pltpu.VMEM((SZ2,), jnp.int32), pltpu.VMEM((SZ2,), jnp.int32),
            pltpu.VMEM((SZ2,), jnp.int32), pltpu.VMEM((SZ2,), jnp.int32),
            pltpu.VMEM((SZ2, DFP), f32), pltpu.VMEM((SZ2, DFP), f32),
            pltpu.VMEM((SZ2, DFP), f32), pltpu.VMEM((SZ2, DFP), f32),
            pltpu.VMEM((SZ2, DFP), f32), pltpu.VMEM((SZ2, DFP), f32),
            pltpu.SemaphoreType.DMA, pltpu.SemaphoreType.DMA,
            pltpu.SemaphoreType.DMA, pltpu.SemaphoreType.DMA,
            pltpu.SemaphoreType.DMA, pltpu.SemaphoreType.DMA,
        ],
    )(a_tab, b_tab, src, dest)


_EC = 2000


def _edge_body(g_ref, e_ref, w_e_e_ref, w_e_o_ref, we2_e_ref, we2_o_ref,
               be2_ref, out_ref):
    f32 = jnp.float32
    gi = lax.bitcast_convert_type(g_ref[...], jnp.int32)
    he = lax.bitcast_convert_type(gi << 16, f32)
    ho = lax.bitcast_convert_type(gi & jnp.int32(-65536), f32)
    eh = e_ref[...]
    h1 = he + jnp.dot(eh, w_e_e_ref[...], preferred_element_type=f32)
    h2 = ho + jnp.dot(eh, w_e_o_ref[...], preferred_element_type=f32)
    out_ref[...] = (
        jnp.dot(jnp.maximum(h1, 0.0), we2_e_ref[...],
                preferred_element_type=f32)
        + jnp.dot(jnp.maximum(h2, 0.0), we2_o_ref[...],
                  preferred_element_type=f32)
        + be2_ref[...])


def _edge_mlp(g, e, w_e_e, w_e_o, we2_e, we2_o, be2):
    return pl.pallas_call(
        _edge_body,
        grid=(E // _EC,),
        in_specs=[
            pl.BlockSpec((_EC, DFP), lambda i: (i, 0)),
            pl.BlockSpec((_EC, DE), lambda i: (i, 0)),
            pl.BlockSpec((DE, DFP), lambda i: (0, 0)),
            pl.BlockSpec((DE, DFP), lambda i: (0, 0)),
            pl.BlockSpec((DFP, DE), lambda i: (0, 0)),
            pl.BlockSpec((DFP, DE), lambda i: (0, 0)),
            pl.BlockSpec((1, DE), lambda i: (0, 0)),
        ],
        out_specs=pl.BlockSpec((_EC, DE), lambda i: (i, 0)),
        out_shape=jax.ShapeDtypeStruct((E, DE), jnp.float32),
    )(g, e, w_e_e, w_e_o, we2_e, we2_o, be2)


_NH = N // 2


def _scatter_body(enew_hbm, dest_hbm, agg_hbm, deg_hbm,
                  ix0, ix1, rv0, rv1, acc_v, deg_v, s0, s1, si0, si1):
    cid = lax.axis_index("c")
    sid = lax.axis_index("s")
    wid = sid * NC + cid

    lane = lax.broadcasted_iota(jnp.int32, (16,), 0)
    zero16 = jnp.zeros((16,), jnp.float32)
    one0 = jnp.where(lane == 0, 1.0, 0.0).astype(jnp.float32)
    m0 = lane == 0

    nc = jnp.where(wid < NCHUNK - (NCHUNK // NW) * NW,
                   NCHUNK // NW + 1, NCHUNK // NW)

    @pl.loop(0, N // 16)
    def _(i):
        deg_v[pl.ds(i * 16, 16)] = zero16

    def start(j, ix, rv, s, si):
        off = (wid + j * NW) * SZ
        pltpu.async_copy(dest_hbm.at[pl.ds(off, SZ)], ix, si)
        pltpu.async_copy(enew_hbm.at[pl.ds(off, SZ)], rv, s)

    def wait(j, ix, rv, s, si):
        off = (wid + j * NW) * SZ
        pltpu.make_async_copy(dest_hbm.at[pl.ds(off, SZ)], ix, si).wait()
        pltpu.make_async_copy(enew_hbm.at[pl.ds(off, SZ)], rv, s).wait()

    for half in range(2):
        lo = half * _NH

        @pl.loop(0, _NH * DE // 16)
        def _(i):
            acc_v[pl.ds(i * 16, 16)] = zero16

        def process(ix, rv):
            @plsc.parallel_loop(0, SZ // 16, unroll=2)
            def _(gi):
                dvec = ix[pl.ds(gi * 16, 16)]
                ddv = dvec - lo
                addrv = ddv * DE
                inrv = jnp.where(
                    jnp.logical_and(ddv >= 0, ddv < _NH), 1, 0
                ).astype(jnp.int32)
                for k in range(16):
                    sel = jnp.full((16,), k, jnp.int32)
                    if half == 0:
                        bd = dvec.at[sel].get(mode="promise_in_bounds")
                        plsc.addupdate_scatter(deg_v, [bd], one0, mask=m0)
                    bad = addrv.at[sel].get(mode="promise_in_bounds")
                    bin_ = inrv.at[sel].get(mode="promise_in_bounds")
                    msk = bin_ != 0
                    adr = jnp.where(msk, bad, 0) + lane
                    plsc.addupdate_scatter(
                        acc_v, [adr], rv[gi * 16 + k, :], mask=msk)

        start(0, ix0, rv0, s0, si0)

        @pl.loop(0, (nc + 1) // 2)
        def _(jj):
            j0 = 2 * jj
            j1 = j0 + 1

            @pl.when(j1 < nc)
            def _():
                start(j1, ix1, rv1, s1, si1)
            wait(j0, ix0, rv0, s0, si0)
            process(ix0, rv0)

            @pl.when(j0 + 2 < nc)
            def _():
                start(j0 + 2, ix0, rv0, s0, si0)

            @pl.when(j1 < nc)
            def _():
                wait(j1, ix1, rv1, s1, si1)
                process(ix1, rv1)

        pltpu.sync_copy(acc_v, agg_hbm.at[pl.ds((wid * N + lo) * DE, _NH * DE)])

    pltpu.sync_copy(deg_v, deg_hbm.at[pl.ds(wid * N, N)])


def _scatter(e_new, dest):
    mesh = plsc.VectorSubcoreMesh(core_axis_name="c", subcore_axis_name="s")
    return pl.kernel(
        _scatter_body,
        out_type=[jax.ShapeDtypeStruct((NW * N * DE,), jnp.float32),
                  jax.ShapeDtypeStruct((NW * N,), jnp.float32)],
        mesh=mesh,
        compiler_params=pltpu.CompilerParams(needs_layout_passes=False),
        scratch_types=[
            pltpu.VMEM((SZ,), jnp.int32), pltpu.VMEM((SZ,), jnp.int32),
            pltpu.VMEM((SZ, DE), jnp.float32), pltpu.VMEM((SZ, DE), jnp.float32),
            pltpu.VMEM((_NH * DE,), jnp.float32),
            pltpu.VMEM((N,), jnp.float32),
            pltpu.SemaphoreType.DMA, pltpu.SemaphoreType.DMA,
            pltpu.SemaphoreType.DMA, pltpu.SemaphoreType.DMA,
        ],
    )(e_new, dest)


def _reduce_body(p_ref, o_ref):
    o_ref[...] = jnp.sum(p_ref[...], axis=0)


def _reduce_agg(parts):
    rows = N * DE // 128
    return pl.pallas_call(
        _reduce_body,
        out_shape=jax.ShapeDtypeStruct((rows, 128), jnp.float32),
    )(parts)


def _reduce_deg(parts):
    return pl.pallas_call(
        _reduce_body,
        out_shape=jax.ShapeDtypeStruct((N // 80, 80), jnp.float32),
    )(parts)


def _finale_body(x_ref, batch_ref, u_ref, agg_ref, deg_ref,
                 wn1x_ref, wn1e_ref, wn1u_ref, bn1_ref, wn2_ref, bn2_ref,
                 wg1u_ref, wg1g_ref, bg1_ref, wg2_ref, bg2_ref,
                 xout_ref, uout_ref):
    f32 = jnp.float32
    x = x_ref[...]
    u = u_ref[...]
    oh = (batch_ref[...] == lax.broadcasted_iota(jnp.int32, (N, B), 1)
          ).astype(f32)
    agg = agg_ref[...] / jnp.clip(deg_ref[...], 1.0, None)
    u2 = jnp.dot(u, wn1u_ref[...], preferred_element_type=f32)
    nh = (jnp.dot(x, wn1x_ref[...], preferred_element_type=f32)
          + jnp.dot(agg, wn1e_ref[...], preferred_element_type=f32)
          + jnp.dot(oh, u2, preferred_element_type=f32)
          + bn1_ref[...])
    xn = (jnp.dot(jnp.maximum(nh, 0.0), wn2_ref[...],
                  preferred_element_type=f32) + bn2_ref[...])
    xout_ref[...] = xn
    dn = (((0,), (0,)), ((), ()))
    gsum = lax.dot_general(oh, xn, dn, preferred_element_type=f32)
    gcnt = lax.dot_general(oh, jnp.ones((N, 1), f32), dn,
                           preferred_element_type=f32)
    gmean = gsum / jnp.clip(gcnt, 1.0, None)
    gh = (jnp.dot(u, wg1u_ref[...], preferred_element_type=f32)
          + jnp.dot(gmean, wg1g_ref[...], preferred_element_type=f32)
          + bg1_ref[...])
    uout_ref[...] = (jnp.dot(jnp.maximum(gh, 0.0), wg2_ref[...],
                             preferred_element_type=f32) + bg2_ref[...])


def _finale(x, batch2d, u, agg_parts, deg_parts,
            wn1x, wn1e, wn1u, bn1, wn2, bn2, wg1u, wg1g, bg1, wg2, bg2):
    return pl.pallas_call(
        _finale_body,
        out_shape=[jax.ShapeDtypeStruct((N, DF), jnp.float32),
                   jax.ShapeDtypeStruct((B, DU), jnp.float32)],
    )(x, batch2d, u, agg_parts, deg_parts,
      wn1x, wn1e, wn1u, bn1, wn2, bn2, wg1u, wg1g, bg1, wg2, bg2)


def kernel(x, edge_index, e, u, batch,
           We1, be1, We2, be2, Wn1, bn1, Wn2, bn2, Wg1, bg1, Wg2, bg2):
    assert x.shape == (N, DF) and edge_index.shape == (2, E)
    assert e.shape == (E, DE) and u.shape == (B, DU) and batch.shape == (N,)

    src = edge_index[0].astype(jnp.int32)
    dest = edge_index[1].astype(jnp.int32)
    batch2d = batch.astype(jnp.int32).reshape(N, 1)

    a_tab, b_tab = _tabs(x, batch2d, u,
                         We1[:DF], We1[DF:2 * DF], We1[2 * DF + DE:],
                         be1.reshape(1, H))

    def to_packed(t):
        return lax.bitcast_convert_type(
            t.astype(jnp.bfloat16).reshape(N, DFP, 2), jnp.float32)

    g = _gather(to_packed(a_tab), to_packed(b_tab), src, dest)
    we1e = We1[2 * DF:2 * DF + DE]
    e_new = _edge_mlp(g, e, we1e[:, 0::2], we1e[:, 1::2],
                      We2[0::2], We2[1::2], be2.reshape(1, DE))
    agg_parts, deg_parts = _scatter(e_new, dest)
    agg = _reduce_agg(agg_parts.reshape(NW, N * DE // 128, 128)).reshape(N, DE)
    deg = _reduce_deg(deg_parts.reshape(NW, N // 80, 80)).reshape(N, 1)
    x_new, u_new = _finale(
        x, batch2d, u, agg, deg,
        Wn1[:DF], Wn1[DF:DF + DE], Wn1[DF + DE:], bn1.reshape(1, H),
        Wn2, bn2.reshape(1, DF),
        Wg1[:DU], Wg1[DU:], bg1.reshape(1, H), Wg2, bg2.reshape(1, DU))
    return (x_new, e_new, u_new)


# speedup vs baseline: 1.2818x; 1.2818x over previous
import functools

import jax
import jax.numpy as jnp
import numpy as np
from jax import lax
from jax.experimental import pallas as pl
from jax.experimental.pallas import tpu as pltpu
from jax.experimental.pallas import tpu_sc as plsc

N = 10000
E = 320000
B = 64
DF = 128
DE = 16
DU = 32
H = 128

NC = 2
NS = 16
NW = NC * NS
DFP = DF // 2
SZ = 128
NCHUNK = E // SZ


def _tabs_body(x_ref, batch_ref, u_ref, w_src_ref, w_dst_ref, w_u_ref, be1_ref,
               a_ref, b_ref):
    x = x_ref[...]
    u1 = jnp.dot(u_ref[...], w_u_ref[...], preferred_element_type=jnp.float32)
    oh = (batch_ref[...] == lax.broadcasted_iota(jnp.int32, (N, B), 1)
          ).astype(jnp.float32)
    a_ref[...] = (jnp.dot(x, w_src_ref[...], preferred_element_type=jnp.float32)
                  + jnp.dot(oh, u1, preferred_element_type=jnp.float32)
                  + be1_ref[...])
    b_ref[...] = jnp.dot(x, w_dst_ref[...], preferred_element_type=jnp.float32)


def _tabs(x, batch2d, u, w_src, w_dst, w_u, be1):
    return pl.pallas_call(
        _tabs_body,
        out_shape=[jax.ShapeDtypeStruct((N, DF), jnp.float32),
                   jax.ShapeDtypeStruct((N, DF), jnp.float32)],
    )(x, batch2d, u, w_src, w_dst, w_u, be1)


SZ2 = 80
EPT = E // NW
CPT = EPT // SZ2


def _gather_body(a_hbm, b_hbm, src_hbm, dest_hbm, g_hbm,
                 isrc, idst, ra0, rb0, ra1, rb1, ob0, ob1,
                 sa0, sb0, sa1, sb1, so0, so1):
    cid = lax.axis_index("c")
    sid = lax.axis_index("s")
    wid = sid * NC + cid
    base = wid * EPT

    pltpu.sync_copy(src_hbm.at[pl.ds(base, EPT)], isrc)
    pltpu.sync_copy(dest_hbm.at[pl.ds(base, EPT)], idst)

    def start(j, ra, rb, sa, sb):
        sl = pl.ds(j * SZ2, SZ2)
        pltpu.async_copy(a_hbm.at[isrc.at[sl]], ra, sa)
        pltpu.async_copy(b_hbm.at[idst.at[sl]], rb, sb)

    def wait_gathers(j, ra, rb, sa, sb):
        sl = pl.ds(j * SZ2, SZ2)
        pltpu.make_async_copy(a_hbm.at[isrc.at[sl]], ra, sa).wait()
        pltpu.make_async_copy(b_hbm.at[idst.at[sl]], rb, sb).wait()

    def add_into(ra, rb, ob):
        @plsc.parallel_loop(0, SZ2, unroll=4)
        def _(i):
            for k in range(DF // 32):
                s1 = ra[i, pl.ds(k * 32, 16)] + rb[i, pl.ds(k * 32, 16)]
                s2 = (ra[i, pl.ds(k * 32 + 16, 16)]
                      + rb[i, pl.ds(k * 32 + 16, 16)])
                w = plsc.pack(s1, s2, format=plsc.PackFormat.INTERLEAVED)
                ob[i, pl.ds(k * 16, 16)] = plsc.bitcast(w, jnp.float32)

    def out_start(j, ob, so):
        pltpu.async_copy(ob, g_hbm.at[pl.ds(base + j * SZ2, SZ2)], so)

    def out_wait(j, ob, so):
        pltpu.make_async_copy(ob, g_hbm.at[pl.ds(base + j * SZ2, SZ2)],
                              so).wait()

    start(0, ra0, rb0, sa0, sb0)

    @pl.loop(0, CPT // 2)
    def _(jj):
        j0 = 2 * jj
        j1 = j0 + 1
        start(j1, ra1, rb1, sa1, sb1)
        wait_gathers(j0, ra0, rb0, sa0, sb0)

        @pl.when(jj > 0)
        def _():
            out_wait(j0 - 2, ob0, so0)
        add_into(ra0, rb0, ob0)
        out_start(j0, ob0, so0)
        start(j0 + 2, ra0, rb0, sa0, sb0)
        wait_gathers(j1, ra1, rb1, sa1, sb1)

        @pl.when(jj > 0)
        def _():
            out_wait(j1 - 2, ob1, so1)
        add_into(ra1, rb1, ob1)
        out_start(j1, ob1, so1)

    wait_gathers(CPT - 1, ra0, rb0, sa0, sb0)
    out_wait(CPT - 3, ob0, so0)
    add_into(ra0, rb0, ob0)
    out_start(CPT - 1, ob0, so0)
    out_wait(CPT - 2, ob1, so1)
    out_wait(CPT - 1, ob0, so0)


def _gather(a_tab, b_tab, src, dest):
    mesh = plsc.VectorSubcoreMesh(core_axis_name="c", subcore_axis_name="s")
    f32 = jnp.float32
    return pl.kernel(
        _gather_body,
        out_type=jax.ShapeDtypeStruct((E, DFP), f32),
        mesh=mesh,
        compiler_params=pltpu.CompilerParams(needs_layout_passes=False),
        scratch_types=[
            pltpu.VMEM((EPT,), jnp.int32), pltpu.VMEM((EPT,), jnp.int32),
            pltpu.VMEM((SZ2, DF), f32), pltpu.VMEM((SZ2, DF), f32),
            pltpu.VMEM((SZ2, DF), f32), pltpu.VMEM((SZ2, DF), f32),
            pltpu.VMEM((SZ2, DFP), f32), pltpu.VMEM((SZ2, DFP), f32),
            pltpu.SemaphoreType.DMA, pltpu.SemaphoreType.DMA,
            pltpu.SemaphoreType.DMA, pltpu.SemaphoreType.DMA,
            pltpu.SemaphoreType.DMA, pltpu.SemaphoreType.DMA,
        ],
    )(a_tab, b_tab, src, dest)


_EC = 2000


def _edge_body(g_ref, e_ref, w_e_e_ref, w_e_o_ref, we2_e_ref, we2_o_ref,
               be2_ref, out_ref):
    f32 = jnp.float32
    gi = lax.bitcast_convert_type(g_ref[...], jnp.int32)
    he = lax.bitcast_convert_type(gi << 16, f32)
    ho = lax.bitcast_convert_type(gi & jnp.int32(-65536), f32)
    eh = e_ref[...]
    h1 = he + jnp.dot(eh, w_e_e_ref[...], preferred_element_type=f32)
    h2 = ho + jnp.dot(eh, w_e_o_ref[...], preferred_element_type=f32)
    out_ref[...] = (
        jnp.dot(jnp.maximum(h1, 0.0), we2_e_ref[...],
                preferred_element_type=f32)
        + jnp.dot(jnp.maximum(h2, 0.0), we2_o_ref[...],
                  preferred_element_type=f32)
        + be2_ref[...])


def _edge_mlp(g, e, w_e_e, w_e_o, we2_e, we2_o, be2):
    return pl.pallas_call(
        _edge_body,
        grid=(E // _EC,),
        in_specs=[
            pl.BlockSpec((_EC, DFP), lambda i: (i, 0)),
            pl.BlockSpec((_EC, DE), lambda i: (i, 0)),
            pl.BlockSpec((DE, DFP), lambda i: (0, 0)),
            pl.BlockSpec((DE, DFP), lambda i: (0, 0)),
            pl.BlockSpec((DFP, DE), lambda i: (0, 0)),
            pl.BlockSpec((DFP, DE), lambda i: (0, 0)),
            pl.BlockSpec((1, DE), lambda i: (0, 0)),
        ],
        out_specs=pl.BlockSpec((_EC, DE), lambda i: (i, 0)),
        out_shape=jax.ShapeDtypeStruct((E, DE), jnp.float32),
    )(g, e, w_e_e, w_e_o, we2_e, we2_o, be2)


_NH = N // 2


def _scatter_body(enew_hbm, dest_hbm, agg_hbm, deg_hbm,
                  ixf, rv0, rv1, acc_v, deg_v, s0, s1):
    cid = lax.axis_index("c")
    sid = lax.axis_index("s")
    wid = sid * NC + cid
    base = wid * EPT

    lane = lax.broadcasted_iota(jnp.int32, (16,), 0)
    zero16 = jnp.zeros((16,), jnp.float32)
    one0 = jnp.where(lane == 0, 1.0, 0.0).astype(jnp.float32)
    m0 = lane == 0

    pltpu.sync_copy(dest_hbm.at[pl.ds(base, EPT)], ixf)

    @pl.loop(0, N // 16)
    def _(i):
        deg_v[pl.ds(i * 16, 16)] = zero16

    def start(j, rv, s):
        pltpu.async_copy(enew_hbm.at[pl.ds(base + j * SZ2, SZ2)], rv, s)

    def wait(j, rv, s):
        pltpu.make_async_copy(enew_hbm.at[pl.ds(base + j * SZ2, SZ2)],
                              rv, s).wait()

    for half in range(2):
        lo = half * _NH

        @pl.loop(0, _NH * DE // 16)
        def _(i):
            acc_v[pl.ds(i * 16, 16)] = zero16

        def process(j, rv):
            @plsc.parallel_loop(0, SZ2 // 16, unroll=2)
            def _(gi):
                dvec = ixf[pl.ds(j * SZ2 + gi * 16, 16)]
                ddv = dvec - lo
                addrv = ddv * DE
                inrv = jnp.where(
                    jnp.logical_and(ddv >= 0, ddv < _NH), 1, 0
                ).astype(jnp.int32)
                for k in range(16):
                    sel = jnp.full((16,), k, jnp.int32)
                    if half == 0:
                        bd = dvec.at[sel].get(mode="promise_in_bounds")
                        plsc.addupdate_scatter(deg_v, [bd], one0, mask=m0)
                    bad = addrv.at[sel].get(mode="promise_in_bounds")
                    bin_ = inrv.at[sel].get(mode="promise_in_bounds")
                    msk = bin_ != 0
                    adr = jnp.where(msk, bad, 0) + lane
                    plsc.addupdate_scatter(
                        acc_v, [adr], rv[gi * 16 + k, :], mask=msk)

        start(0, rv0, s0)

        @pl.loop(0, CPT // 2)
        def _(jj):
            j0 = 2 * jj
            j1 = j0 + 1
            start(j1, rv1, s1)
            wait(j0, rv0, s0)
            process(j0, rv0)
            start(j0 + 2, rv0, s0)
            wait(j1, rv1, s1)
            process(j1, rv1)

        wait(CPT - 1, rv0, s0)
        process(CPT - 1, rv0)

        pltpu.sync_copy(acc_v, agg_hbm.at[pl.ds((wid * N + lo) * DE, _NH * DE)])

    pltpu.sync_copy(deg_v, deg_hbm.at[pl.ds(wid * N, N)])


def _scatter(e_new, dest):
    mesh = plsc.VectorSubcoreMesh(core_axis_name="c", subcore_axis_name="s")
    return pl.kernel(
        _scatter_body,
        out_type=[jax.ShapeDtypeStruct((NW * N * DE,), jnp.float32),
                  jax.ShapeDtypeStruct((NW * N,), jnp.float32)],
        mesh=mesh,
        compiler_params=pltpu.CompilerParams(needs_layout_passes=False),
        scratch_types=[
            pltpu.VMEM((EPT,), jnp.int32),
            pltpu.VMEM((SZ2, DE), jnp.float32),
            pltpu.VMEM((SZ2, DE), jnp.float32),
            pltpu.VMEM((_NH * DE,), jnp.float32),
            pltpu.VMEM((N,), jnp.float32),
            pltpu.SemaphoreType.DMA, pltpu.SemaphoreType.DMA,
        ],
    )(e_new, dest)


def _reduce_body(p_ref, o_ref):
    o_ref[...] = jnp.sum(p_ref[...], axis=0)


def _reduce_agg(parts):
    rows = N * DE // 128
    return pl.pallas_call(
        _reduce_body,
        out_shape=jax.ShapeDtypeStruct((rows, 128), jnp.float32),
    )(parts)


def _reduce_deg(parts):
    return pl.pallas_call(
        _reduce_body,
        out_shape=jax.ShapeDtypeStruct((N // 80, 80), jnp.float32),
    )(parts)


def _finale_body(x_ref, batch_ref, u_ref, agg_ref, deg_ref,
                 wn1x_ref, wn1e_ref, wn1u_ref, bn1_ref, wn2_ref, bn2_ref,
                 wg1u_ref, wg1g_ref, bg1_ref, wg2_ref, bg2_ref,
                 xout_ref, uout_ref):
    f32 = jnp.float32
    x = x_ref[...]
    u = u_ref[...]
    oh = (batch_ref[...] == lax.broadcasted_iota(jnp.int32, (N, B), 1)
          ).astype(f32)
    agg = agg_ref[...] / jnp.clip(deg_ref[...], 1.0, None)
    u2 = jnp.dot(u, wn1u_ref[...], preferred_element_type=f32)
    nh = (jnp.dot(x, wn1x_ref[...], preferred_element_type=f32)
          + jnp.dot(agg, wn1e_ref[...], preferred_element_type=f32)
          + jnp.dot(oh, u2, preferred_element_type=f32)
          + bn1_ref[...])
    xn = (jnp.dot(jnp.maximum(nh, 0.0), wn2_ref[...],
                  preferred_element_type=f32) + bn2_ref[...])
    xout_ref[...] = xn
    dn = (((0,), (0,)), ((), ()))
    gsum = lax.dot_general(oh, xn, dn, preferred_element_type=f32)
    gcnt = lax.dot_general(oh, jnp.ones((N, 1), f32), dn,
                           preferred_element_type=f32)
    gmean = gsum / jnp.clip(gcnt, 1.0, None)
    gh = (jnp.dot(u, wg1u_ref[...], preferred_element_type=f32)
          + jnp.dot(gmean, wg1g_ref[...], preferred_element_type=f32)
          + bg1_ref[...])
    uout_ref[...] = (jnp.dot(jnp.maximum(gh, 0.0), wg2_ref[...],
                             preferred_element_type=f32) + bg2_ref[...])


def _finale(x, batch2d, u, agg_parts, deg_parts,
            wn1x, wn1e, wn1u, bn1, wn2, bn2, wg1u, wg1g, bg1, wg2, bg2):
    return pl.pallas_call(
        _finale_body,
        out_shape=[jax.ShapeDtypeStruct((N, DF), jnp.float32),
                   jax.ShapeDtypeStruct((B, DU), jnp.float32)],
    )(x, batch2d, u, agg_parts, deg_parts,
      wn1x, wn1e, wn1u, bn1, wn2, bn2, wg1u, wg1g, bg1, wg2, bg2)


def kernel(x, edge_index, e, u, batch,
           We1, be1, We2, be2, Wn1, bn1, Wn2, bn2, Wg1, bg1, Wg2, bg2):
    assert x.shape == (N, DF) and edge_index.shape == (2, E)
    assert e.shape == (E, DE) and u.shape == (B, DU) and batch.shape == (N,)

    src = edge_index[0].astype(jnp.int32)
    dest = edge_index[1].astype(jnp.int32)
    batch2d = batch.astype(jnp.int32).reshape(N, 1)

    a_tab, b_tab = _tabs(x, batch2d, u,
                         We1[:DF], We1[DF:2 * DF], We1[2 * DF + DE:],
                         be1.reshape(1, H))

    g = _gather(a_tab, b_tab, src, dest)
    perm_e = np.concatenate([np.arange(32 * k, 32 * k + 16) for k in range(4)])
    perm_o = perm_e + 16
    we1e = We1[2 * DF:2 * DF + DE]
    e_new = _edge_mlp(g, e, we1e[:, perm_e], we1e[:, perm_o],
                      We2[perm_e], We2[perm_o], be2.reshape(1, DE))
    agg_parts, deg_parts = _scatter(e_new, dest)
    agg = _reduce_agg(agg_parts.reshape(NW, N * DE // 128, 128)).reshape(N, DE)
    deg = _reduce_deg(deg_parts.reshape(NW, N // 80, 80)).reshape(N, 1)
    x_new, u_new = _finale(
        x, batch2d, u, agg, deg,
        Wn1[:DF], Wn1[DF:DF + DE], Wn1[DF + DE:], bn1.reshape(1, H),
        Wn2, bn2.reshape(1, DF),
        Wg1[:DU], Wg1[DU:], bg1.reshape(1, H), Wg2, bg2.reshape(1, DU))
    return (x_new, e_new, u_new)


# speedup vs baseline: 1.3908x; 1.0850x over previous
import functools

import jax
import jax.numpy as jnp
import numpy as np
from jax import lax
from jax.experimental import pallas as pl
from jax.experimental.pallas import tpu as pltpu
from jax.experimental.pallas import tpu_sc as plsc

N = 10000
E = 320000
B = 64
DF = 128
DE = 16
DU = 32
H = 128

NC = 2
NS = 16
NW = NC * NS
DFP = DF // 2
SZ = 128
NCHUNK = E // SZ


def _tabs_body(x_ref, batch_ref, u_ref, w_src_ref, w_dst_ref, w_u_ref, be1_ref,
               a_ref, b_ref):
    x = x_ref[...]
    u1 = jnp.dot(u_ref[...], w_u_ref[...], preferred_element_type=jnp.float32)
    oh = (batch_ref[...] == lax.broadcasted_iota(jnp.int32, (N, B), 1)
          ).astype(jnp.float32)
    a_ref[...] = (jnp.dot(x, w_src_ref[...], preferred_element_type=jnp.float32)
                  + jnp.dot(oh, u1, preferred_element_type=jnp.float32)
                  + be1_ref[...])
    b_ref[...] = jnp.dot(x, w_dst_ref[...], preferred_element_type=jnp.float32)


def _tabs(x, batch2d, u, w_src, w_dst, w_u, be1):
    return pl.pallas_call(
        _tabs_body,
        out_shape=[jax.ShapeDtypeStruct((N, DF), jnp.float32),
                   jax.ShapeDtypeStruct((N, DF), jnp.float32)],
    )(x, batch2d, u, w_src, w_dst, w_u, be1)


SZ2 = 80
EPT = E // NW
CPT = EPT // SZ2


def _gather_body(a_hbm, b_hbm, src_hbm, dest_hbm, g_hbm,
                 isrc, idst, ra0, rb0, ra1, rb1, ob0, ob1,
                 sa0, sb0, sa1, sb1, so0, so1):
    cid = lax.axis_index("c")
    sid = lax.axis_index("s")
    wid = sid * NC + cid
    base = wid * EPT

    pltpu.sync_copy(src_hbm.at[pl.ds(base, EPT)], isrc)
    pltpu.sync_copy(dest_hbm.at[pl.ds(base, EPT)], idst)

    def start(j, ra, rb, sa, sb):
        sl = pl.ds(j * SZ2, SZ2)
        pltpu.async_copy(a_hbm.at[isrc.at[sl]], ra, sa)
        pltpu.async_copy(b_hbm.at[idst.at[sl]], rb, sb)

    def wait_gathers(j, ra, rb, sa, sb):
        sl = pl.ds(j * SZ2, SZ2)
        pltpu.make_async_copy(a_hbm.at[isrc.at[sl]], ra, sa).wait()
        pltpu.make_async_copy(b_hbm.at[idst.at[sl]], rb, sb).wait()

    def add_into(ra, rb, ob):
        @plsc.parallel_loop(0, SZ2, unroll=4)
        def _(i):
            for k in range(DF // 32):
                s1 = ra[i, pl.ds(k * 32, 16)] + rb[i, pl.ds(k * 32, 16)]
                s2 = (ra[i, pl.ds(k * 32 + 16, 16)]
                      + rb[i, pl.ds(k * 32 + 16, 16)])
                w = plsc.pack(s1, s2, format=plsc.PackFormat.INTERLEAVED)
                ob[i, pl.ds(k * 16, 16)] = plsc.bitcast(w, jnp.float32)

    def out_start(j, ob, so):
        pltpu.async_copy(ob, g_hbm.at[pl.ds(base + j * SZ2, SZ2)], so)

    def out_wait(j, ob, so):
        pltpu.make_async_copy(ob, g_hbm.at[pl.ds(base + j * SZ2, SZ2)],
                              so).wait()

    start(0, ra0, rb0, sa0, sb0)

    @pl.loop(0, CPT // 2)
    def _(jj):
        j0 = 2 * jj
        j1 = j0 + 1
        start(j1, ra1, rb1, sa1, sb1)
        wait_gathers(j0, ra0, rb0, sa0, sb0)

        @pl.when(jj > 0)
        def _():
            out_wait(j0 - 2, ob0, so0)
        add_into(ra0, rb0, ob0)
        out_start(j0, ob0, so0)
        start(j0 + 2, ra0, rb0, sa0, sb0)
        wait_gathers(j1, ra1, rb1, sa1, sb1)

        @pl.when(jj > 0)
        def _():
            out_wait(j1 - 2, ob1, so1)
        add_into(ra1, rb1, ob1)
        out_start(j1, ob1, so1)

    wait_gathers(CPT - 1, ra0, rb0, sa0, sb0)
    out_wait(CPT - 3, ob0, so0)
    add_into(ra0, rb0, ob0)
    out_start(CPT - 1, ob0, so0)
    out_wait(CPT - 2, ob1, so1)
    out_wait(CPT - 1, ob0, so0)


def _gather(a_tab, b_tab, src, dest):
    mesh = plsc.VectorSubcoreMesh(core_axis_name="c", subcore_axis_name="s")
    f32 = jnp.float32
    return pl.kernel(
        _gather_body,
        out_type=jax.ShapeDtypeStruct((E, DFP), f32),
        mesh=mesh,
        compiler_params=pltpu.CompilerParams(needs_layout_passes=False),
        scratch_types=[
            pltpu.VMEM((EPT,), jnp.int32), pltpu.VMEM((EPT,), jnp.int32),
            pltpu.VMEM((SZ2, DF), f32), pltpu.VMEM((SZ2, DF), f32),
            pltpu.VMEM((SZ2, DF), f32), pltpu.VMEM((SZ2, DF), f32),
            pltpu.VMEM((SZ2, DFP), f32), pltpu.VMEM((SZ2, DFP), f32),
            pltpu.SemaphoreType.DMA, pltpu.SemaphoreType.DMA,
            pltpu.SemaphoreType.DMA, pltpu.SemaphoreType.DMA,
            pltpu.SemaphoreType.DMA, pltpu.SemaphoreType.DMA,
        ],
    )(a_tab, b_tab, src, dest)


_EC = 8000


def _edge_body(g_ref, e_ref, w_e_e_ref, w_e_o_ref, we2_e_ref, we2_o_ref,
               be2_ref, out_ref):
    f32 = jnp.float32
    gi = lax.bitcast_convert_type(g_ref[...], jnp.int32)
    he = lax.bitcast_convert_type(gi << 16, f32)
    ho = lax.bitcast_convert_type(gi & jnp.int32(-65536), f32)
    eh = e_ref[...]
    h1 = he + jnp.dot(eh, w_e_e_ref[...], preferred_element_type=f32)
    h2 = ho + jnp.dot(eh, w_e_o_ref[...], preferred_element_type=f32)
    out_ref[...] = (
        jnp.dot(jnp.maximum(h1, 0.0), we2_e_ref[...],
                preferred_element_type=f32)
        + jnp.dot(jnp.maximum(h2, 0.0), we2_o_ref[...],
                  preferred_element_type=f32)
        + be2_ref[...])


def _edge_mlp(g, e, w_e_e, w_e_o, we2_e, we2_o, be2):
    return pl.pallas_call(
        _edge_body,
        grid=(E // _EC,),
        in_specs=[
            pl.BlockSpec((_EC, DFP), lambda i: (i, 0)),
            pl.BlockSpec((_EC, DE), lambda i: (i, 0)),
            pl.BlockSpec((DE, DFP), lambda i: (0, 0)),
            pl.BlockSpec((DE, DFP), lambda i: (0, 0)),
            pl.BlockSpec((DFP, DE), lambda i: (0, 0)),
            pl.BlockSpec((DFP, DE), lambda i: (0, 0)),
            pl.BlockSpec((1, DE), lambda i: (0, 0)),
        ],
        out_specs=pl.BlockSpec((_EC, DE), lambda i: (i, 0)),
        out_shape=jax.ShapeDtypeStruct((E, DE), jnp.float32),
    )(g, e, w_e_e, w_e_o, we2_e, we2_o, be2)


_NH = N // 2


def _scatter_body(enew_hbm, dest_hbm, agg_hbm, deg_hbm,
                  ixf, rv0, rv1, acc_v, deg_v, s0, s1):
    cid = lax.axis_index("c")
    sid = lax.axis_index("s")
    wid = sid * NC + cid
    base = wid * EPT

    lane = lax.broadcasted_iota(jnp.int32, (16,), 0)
    zero16 = jnp.zeros((16,), jnp.float32)
    one0 = jnp.where(lane == 0, 1.0, 0.0).astype(jnp.float32)
    m0 = lane == 0

    pltpu.sync_copy(dest_hbm.at[pl.ds(base, EPT)], ixf)

    @pl.loop(0, N // 16)
    def _(i):
        deg_v[pl.ds(i * 16, 16)] = zero16

    def start(j, rv, s):
        pltpu.async_copy(enew_hbm.at[pl.ds(base + j * SZ2, SZ2)], rv, s)

    def wait(j, rv, s):
        pltpu.make_async_copy(enew_hbm.at[pl.ds(base + j * SZ2, SZ2)],
                              rv, s).wait()

    for half in range(2):
        lo = half * _NH

        @pl.loop(0, _NH * DE // 16)
        def _(i):
            acc_v[pl.ds(i * 16, 16)] = zero16

        def process(j, rv):
            @plsc.parallel_loop(0, SZ2 // 16, unroll=2)
            def _(gi):
                dvec = ixf[pl.ds(j * SZ2 + gi * 16, 16)]
                ddv = dvec - lo
                addrv = ddv * DE
                inrv = jnp.where(
                    jnp.logical_and(ddv >= 0, ddv < _NH), 1, 0
                ).astype(jnp.int32)
                for k in range(16):
                    sel = jnp.full((16,), k, jnp.int32)
                    if half == 0:
                        bd = dvec.at[sel].get(mode="promise_in_bounds")
                        plsc.addupdate_scatter(deg_v, [bd], one0, mask=m0)
                    bad = addrv.at[sel].get(mode="promise_in_bounds")
                    bin_ = inrv.at[sel].get(mode="promise_in_bounds")
                    msk = bin_ != 0
                    adr = jnp.where(msk, bad, 0) + lane
                    plsc.addupdate_scatter(
                        acc_v, [adr], rv[gi * 16 + k, :], mask=msk)

        start(0, rv0, s0)

        @pl.loop(0, CPT // 2)
        def _(jj):
            j0 = 2 * jj
            j1 = j0 + 1
            start(j1, rv1, s1)
            wait(j0, rv0, s0)
            process(j0, rv0)
            start(j0 + 2, rv0, s0)
            wait(j1, rv1, s1)
            process(j1, rv1)

        wait(CPT - 1, rv0, s0)
        process(CPT - 1, rv0)

        pltpu.sync_copy(acc_v, agg_hbm.at[pl.ds((wid * N + lo) * DE, _NH * DE)])

    pltpu.sync_copy(deg_v, deg_hbm.at[pl.ds(wid * N, N)])


def _scatter(e_new, dest):
    mesh = plsc.VectorSubcoreMesh(core_axis_name="c", subcore_axis_name="s")
    return pl.kernel(
        _scatter_body,
        out_type=[jax.ShapeDtypeStruct((NW * N * DE,), jnp.float32),
                  jax.ShapeDtypeStruct((NW * N,), jnp.float32)],
        mesh=mesh,
        compiler_params=pltpu.CompilerParams(needs_layout_passes=False),
        scratch_types=[
            pltpu.VMEM((EPT,), jnp.int32),
            pltpu.VMEM((SZ2, DE), jnp.float32),
            pltpu.VMEM((SZ2, DE), jnp.float32),
            pltpu.VMEM((_NH * DE,), jnp.float32),
            pltpu.VMEM((N,), jnp.float32),
            pltpu.SemaphoreType.DMA, pltpu.SemaphoreType.DMA,
        ],
    )(e_new, dest)


def _reduce_body(p_ref, d_ref, oa_ref, od_ref):
    oa_ref[...] = jnp.sum(p_ref[...], axis=0)
    od_ref[...] = jnp.sum(d_ref[...], axis=0)


def _reduce_parts(agg_parts, deg_parts):
    rows = N * DE // 128
    return pl.pallas_call(
        _reduce_body,
        out_shape=[jax.ShapeDtypeStruct((rows, 128), jnp.float32),
                   jax.ShapeDtypeStruct((N // 80, 80), jnp.float32)],
    )(agg_parts, deg_parts)


def _finale_body(x_ref, batch_ref, u_ref, agg_ref, deg_ref,
                 wn1x_ref, wn1e_ref, wn1u_ref, bn1_ref, wn2_ref, bn2_ref,
                 wg1u_ref, wg1g_ref, bg1_ref, wg2_ref, bg2_ref,
                 xout_ref, uout_ref):
    f32 = jnp.float32
    x = x_ref[...]
    u = u_ref[...]
    oh = (batch_ref[...] == lax.broadcasted_iota(jnp.int32, (N, B), 1)
          ).astype(f32)
    agg = agg_ref[...] / jnp.clip(deg_ref[...], 1.0, None)
    u2 = jnp.dot(u, wn1u_ref[...], preferred_element_type=f32)
    nh = (jnp.dot(x, wn1x_ref[...], preferred_element_type=f32)
          + jnp.dot(agg, wn1e_ref[...], preferred_element_type=f32)
          + jnp.dot(oh, u2, preferred_element_type=f32)
          + bn1_ref[...])
    xn = (jnp.dot(jnp.maximum(nh, 0.0), wn2_ref[...],
                  preferred_element_type=f32) + bn2_ref[...])
    xout_ref[...] = xn
    dn = (((0,), (0,)), ((), ()))
    gsum = lax.dot_general(oh, xn, dn, preferred_element_type=f32)
    gcnt = lax.dot_general(oh, jnp.ones((N, 1), f32), dn,
                           preferred_element_type=f32)
    gmean = gsum / jnp.clip(gcnt, 1.0, None)
    gh = (jnp.dot(u, wg1u_ref[...], preferred_element_type=f32)
          + jnp.dot(gmean, wg1g_ref[...], preferred_element_type=f32)
          + bg1_ref[...])
    uout_ref[...] = (jnp.dot(jnp.maximum(gh, 0.0), wg2_ref[...],
                             preferred_element_type=f32) + bg2_ref[...])


def _finale(x, batch2d, u, agg_parts, deg_parts,
            wn1x, wn1e, wn1u, bn1, wn2, bn2, wg1u, wg1g, bg1, wg2, bg2):
    return pl.pallas_call(
        _finale_body,
        out_shape=[jax.ShapeDtypeStruct((N, DF), jnp.float32),
                   jax.ShapeDtypeStruct((B, DU), jnp.float32)],
    )(x, batch2d, u, agg_parts, deg_parts,
      wn1x, wn1e, wn1u, bn1, wn2, bn2, wg1u, wg1g, bg1, wg2, bg2)


def kernel(x, edge_index, e, u, batch,
           We1, be1, We2, be2, Wn1, bn1, Wn2, bn2, Wg1, bg1, Wg2, bg2):
    assert x.shape == (N, DF) and edge_index.shape == (2, E)
    assert e.shape == (E, DE) and u.shape == (B, DU) and batch.shape == (N,)

    src = edge_index[0].astype(jnp.int32)
    dest = edge_index[1].astype(jnp.int32)
    batch2d = batch.astype(jnp.int32).reshape(N, 1)

    a_tab, b_tab = _tabs(x, batch2d, u,
                         We1[:DF], We1[DF:2 * DF], We1[2 * DF + DE:],
                         be1.reshape(1, H))

    g = _gather(a_tab, b_tab, src, dest)
    perm_e = np.concatenate([np.arange(32 * k, 32 * k + 16) for k in range(4)])
    perm_o = perm_e + 16
    we1e = We1[2 * DF:2 * DF + DE]
    e_new = _edge_mlp(g, e, we1e[:, perm_e], we1e[:, perm_o],
                      We2[perm_e], We2[perm_o], be2.reshape(1, DE))
    agg_parts, deg_parts = _scatter(e_new, dest)
    agg_r, deg_r = _reduce_parts(agg_parts.reshape(NW, N * DE // 128, 128),
                                 deg_parts.reshape(NW, N // 80, 80))
    agg = agg_r.reshape(N, DE)
    deg = deg_r.reshape(N, 1)
    x_new, u_new = _finale(
        x, batch2d, u, agg, deg,
        Wn1[:DF], Wn1[DF:DF + DE], Wn1[DF + DE:], bn1.reshape(1, H),
        Wn2, bn2.reshape(1, DF),
        Wg1[:DU], Wg1[DU:], bg1.reshape(1, H), Wg2, bg2.reshape(1, DU))
    return (x_new, e_new, u_new)
